# baseline (speedup 1.0000x reference)
import functools

import jax
import jax.numpy as jnp
from jax import lax
from jax.experimental import pallas as pl
from jax.experimental.pallas import tpu as pltpu
from jax.experimental.pallas import tpu_sc as plsc

B = 16384
D = 64
BLK = 2048
NC = 2
NS = 16
NW = NC * NS


def _sc_gather(poi_table, idx_all):
    n = idx_all.shape[0]
    r = n // NW
    mesh = plsc.VectorSubcoreMesh(
        core_axis_name="c", subcore_axis_name="s", num_cores=NC, num_subcores=NS
    )

    @functools.partial(
        pl.kernel,
        out_type=jax.ShapeDtypeStruct((n, D), jnp.float32),
        mesh=mesh,
        scratch_types=[
            pltpu.VMEM((r,), jnp.int32),
            pltpu.VMEM((r, D), jnp.float32),
            pltpu.SemaphoreType.DMA,
        ],
        compiler_params=pltpu.CompilerParams(use_tc_tiling_on_sc=False),
    )
    def gather_kernel(table_hbm, idx_hbm, out_hbm, idx_v, rows_v, sem):
        wid = lax.axis_index("s") * NC + lax.axis_index("c")
        base = wid * r
        pltpu.sync_copy(idx_hbm.at[pl.ds(base, r)], idx_v)
        pltpu.async_copy(table_hbm.at[idx_v], rows_v, sem).wait()
        pltpu.sync_copy(rows_v, out_hbm.at[pl.ds(base, r)])

    return gather_kernel(poi_table, idx_all)


def _tc_body(h_ref, t_ref, nt_ref,
             time_idx_ref, now_idx_ref, d0_ref, d1_ref, d2_ref, m_idx_ref,
             time_tab_ref, now_tab_ref, day_tab_ref, month_tab_ref,
             wday_ref, bd_ref, ww_ref, bw_ref, out_ref):
    i = pl.program_id(0)

    def onehot_rows(idx, tab_ref):
        ntab = tab_ref.shape[0]
        oh = (idx[:, None] == lax.broadcasted_iota(jnp.int32, (BLK, ntab), 1))
        return lax.dot_general(
            oh.astype(jnp.float32), tab_ref[...], (((1,), (0,)), ((), ())),
            preferred_element_type=jnp.float32, precision=lax.Precision.HIGHEST)

    t_time = onehot_rows(time_idx_ref[...], time_tab_ref)
    r_w_now = onehot_rows(now_idx_ref[...], now_tab_ref)
    r_w_minus = onehot_rows(d0_ref[...], day_tab_ref)
    r_w_curr = onehot_rows(d1_ref[...], day_tab_ref)
    r_w_plus = onehot_rows(d2_ref[...], day_tab_ref)
    e_month = onehot_rows(m_idx_ref[...], month_tab_ref)

    concat_day = jnp.concatenate([r_w_minus, r_w_curr, r_w_plus], axis=1)
    e_day = lax.dot_general(
        concat_day, wday_ref[...], (((1,), (1,)), ((), ())),
        preferred_element_type=jnp.float32,
        precision=lax.Precision.HIGHEST) + bd_ref[...]
    concat_weather = jnp.concatenate([r_w_now, e_day, e_month], axis=1)
    e_w = lax.dot_general(
        concat_weather, ww_ref[...], (((1,), (1,)), ((), ())),
        preferred_element_type=jnp.float32,
        precision=lax.Precision.HIGHEST) + bw_ref[...]

    hr = h_ref[...] + t_time + e_w
    dp = hr - t_ref[...]
    dn = hr - nt_ref[...]
    pos = jnp.sum(dp * dp, axis=1)
    neg = jnp.sum(dn * dn, axis=1)
    part = jnp.sum(jnp.maximum(pos + 1.0 - neg, 0.0))

    @pl.when(i == 0)
    def _():
        out_ref[...] = jnp.zeros_like(out_ref)

    out_ref[...] += part

    @pl.when(i == pl.num_programs(0) - 1)
    def _():
        out_ref[...] = out_ref[...] * (1.0 / B)


def kernel(head_idx, r_time_idx, r_weather_idx, tail_idx, neg_tail_idx,
           r_season_idx, r_day_seq_idx, r_month_idx,
           poi_table, time_table, now_table, day_table, month_table,
           season_table, W_day, b_d, W_w, b_w):
    del r_season_idx, season_table
    idx_all = jnp.concatenate(
        [head_idx, tail_idx, neg_tail_idx]).astype(jnp.int32)
    rows = _sc_gather(poi_table, idx_all)

    def pad16(tab):
        ntab = tab.shape[0]
        if ntab % 8:
            tab = jnp.concatenate(
                [tab, jnp.zeros((16 - ntab, D), tab.dtype)], axis=0)
        return tab

    nb = B // BLK
    grid_spec = pl.GridSpec(
        grid=(nb,),
        in_specs=[
            pl.BlockSpec((BLK, D), lambda i: (i, 0)),
            pl.BlockSpec((BLK, D), lambda i: (i + nb, 0)),
            pl.BlockSpec((BLK, D), lambda i: (i + 2 * nb, 0)),
            pl.BlockSpec((BLK,), lambda i: (i,)),
            pl.BlockSpec((BLK,), lambda i: (i,)),
            pl.BlockSpec((BLK,), lambda i: (i,)),
            pl.BlockSpec((BLK,), lambda i: (i,)),
            pl.BlockSpec((BLK,), lambda i: (i,)),
            pl.BlockSpec((BLK,), lambda i: (i,)),
            pl.BlockSpec((48, D), lambda i: (0, 0)),
            pl.BlockSpec((16, D), lambda i: (0, 0)),
            pl.BlockSpec((16, D), lambda i: (0, 0)),
            pl.BlockSpec((16, D), lambda i: (0, 0)),
            pl.BlockSpec((D, 3 * D), lambda i: (0, 0)),
            pl.BlockSpec((1, D), lambda i: (0, 0)),
            pl.BlockSpec((D, 3 * D), lambda i: (0, 0)),
            pl.BlockSpec((1, D), lambda i: (0, 0)),
        ],
        out_specs=pl.BlockSpec((1, 1), lambda i: (0, 0)),
    )
    out = pl.pallas_call(
        _tc_body,
        grid_spec=grid_spec,
        out_shape=jax.ShapeDtypeStruct((1, 1), jnp.float32),
    )(rows, rows, rows,
      r_time_idx.astype(jnp.int32), r_weather_idx.astype(jnp.int32),
      r_day_seq_idx[:, 0].astype(jnp.int32),
      r_day_seq_idx[:, 1].astype(jnp.int32),
      r_day_seq_idx[:, 2].astype(jnp.int32),
      r_month_idx.astype(jnp.int32),
      time_table, pad16(now_table), pad16(day_table), pad16(month_table),
      W_day, b_d.reshape(1, D), W_w, b_w.reshape(1, D))
    return out[0, 0]
